# baseline (speedup 1.0000x reference)
import functools

import jax
import jax.numpy as jnp
from jax import lax
from jax.experimental import pallas as pl
from jax.experimental.pallas import tpu as pltpu
from jax.experimental.pallas import tpu_sc as plsc

_NUM_EMB = 1024
_EMB_DIM = 256
_GROUP_NUM = 4
_GROUP_DIM = _EMB_DIM // _GROUP_NUM
_CC = 0.25
_DECAY = 0.99
_EPS = 1e-05

_ROWS = 32 * 576
_TILE = 2048
_NTILE = _ROWS // _TILE

_NC = 2
_NS = 16
_NW = _NC * _NS
_RPW = _ROWS // _NW
_CHUNK = 64
_NCHUNK = _RPW // _CHUNK


def _argmin_body(flat_ref, emb_ref, idx_ref, dsum_ref, dwx_ref, *, g):
    f = flat_ref[:, g, :]
    e = emb_ref[...]
    fb = f.astype(jnp.bfloat16)
    eb = e.astype(jnp.bfloat16)
    dot = lax.dot_general(fb, eb, (((1,), (1,)), ((), ())),
                          preferred_element_type=jnp.float32)
    f2 = jnp.sum(f * f, axis=1, keepdims=True)
    e2 = jnp.sum(e * e, axis=1)
    d = f2 + e2[None, :] - 2.0 * dot
    m = jnp.min(d, axis=1, keepdims=True)
    idx = jnp.argmin(d, axis=1).astype(jnp.int32)
    idx_ref[...] = idx[None, None, :].astype(jnp.int32)

    row = lax.broadcasted_iota(jnp.int32, (_NUM_EMB, _TILE), 0)
    encT = (row == idx[None, :]).astype(jnp.bfloat16)
    fx = jnp.concatenate(
        [fb, jnp.ones((_TILE, _GROUP_DIM), jnp.bfloat16)], axis=1)
    stat = lax.dot_general(encT, fx, (((1,), (0,)), ((), ())),
                           preferred_element_type=jnp.float32)

    part = jnp.sum(m.reshape(_TILE // 128, 128), axis=0)

    @pl.when(pl.program_id(0) == 0)
    def _init():
        dsum_ref[...] = jnp.zeros_like(dsum_ref)
        dwx_ref[...] = jnp.zeros_like(dwx_ref)

    dsum_ref[...] += part[None, :]
    dwx_ref[...] += stat


def _argmin_call(g, x4, emb):
    return pl.pallas_call(
        functools.partial(_argmin_body, g=g),
        grid=(_NTILE,),
        in_specs=[
            pl.BlockSpec((_TILE, _GROUP_NUM, _GROUP_DIM), lambda i: (i, 0, 0)),
            pl.BlockSpec((_NUM_EMB, _GROUP_DIM), lambda i: (0, 0)),
        ],
        out_specs=[
            pl.BlockSpec((1, 1, _TILE), lambda i: (i, 0, 0)),
            pl.BlockSpec((1, 128), lambda i: (0, 0)),
            pl.BlockSpec((_NUM_EMB, 2 * _GROUP_DIM), lambda i: (0, 0)),
        ],
        out_shape=[
            jax.ShapeDtypeStruct((_NTILE, 1, _TILE), jnp.int32),
            jax.ShapeDtypeStruct((1, 128), jnp.float32),
            jax.ShapeDtypeStruct((_NUM_EMB, 2 * _GROUP_DIM), jnp.float32),
        ],
    )(x4, emb)


def _gather_body(idx_hbm, emb_hbm, quant_hbm, idx_v, qrows, sem):
    c = lax.axis_index("c")
    s = lax.axis_index("s")
    wid = c * _NS + s
    base = pl.multiple_of(wid * _RPW, _RPW)

    pltpu.sync_copy(idx_hbm.at[wid], idx_v)
    cps = [
        pltpu.async_copy(emb_hbm.at[idx_v.at[j]],
                         qrows.at[pl.ds(j * _CHUNK, _CHUNK)], sem)
        for j in range(_NCHUNK)
    ]
    for cp in cps:
        cp.wait()
    pltpu.sync_copy(qrows, quant_hbm.at[pl.ds(base, _RPW)])


@functools.cache
def _gather_kernel():
    return pl.kernel(
        _gather_body,
        out_type=[
            jax.ShapeDtypeStruct((_ROWS, _GROUP_DIM), jnp.float32),
        ],
        mesh=plsc.VectorSubcoreMesh(core_axis_name="c", subcore_axis_name="s",
                                    num_cores=_NC, num_subcores=_NS),
        compiler_params=pltpu.CompilerParams(use_tc_tiling_on_sc=False),
        scratch_types=[
            pltpu.VMEM((_NCHUNK, _CHUNK), jnp.int32),
            pltpu.VMEM((_RPW, _GROUP_DIM), jnp.float32),
            pltpu.SemaphoreType.DMA,
        ],
    )


def _ema_body(dwx_ref, hc_ref, hdw_ref, dsum_ref,
              emb_out, hc_out, hdw_out, stat_ref, *, counter):
    dwx = dwx_ref[...]
    counts = dwx[:, _GROUP_DIM:_GROUP_DIM + 1]
    dw = dwx[:, :_GROUP_DIM]
    hc = hc_ref[...]
    hc_new = hc - (hc - counts) * (1.0 - _DECAY)
    bias = 1.0 - _DECAY ** counter
    avg_c = hc_new / bias
    n = jnp.sum(avg_c)
    upd_c = (avg_c + _EPS) / (n + _NUM_EMB * _EPS) * n

    hdw = hdw_ref[...]
    hdw_new = hdw - (hdw - dw) * (1.0 - _DECAY)
    avg_dw = hdw_new / bias

    emb_out[...] = avg_dw / upd_c
    hc_out[...] = hc_new
    hdw_out[...] = hdw_new

    avg_p = counts / float(_ROWS)
    perp = jnp.exp(-jnp.sum(avg_p * jnp.log(avg_p + 1e-06)))
    loss = _CC * (jnp.sum(dsum_ref[...]) / float(_ROWS * _GROUP_DIM))
    lane = lax.broadcasted_iota(jnp.int32, (1, 128), 1)
    stat_ref[...] = jnp.where(lane == 0, loss,
                              jnp.where(lane == 1, perp, 0.0))


def _ema_call(counter, dwx, hc, hdw, dsum):
    return pl.pallas_call(
        functools.partial(_ema_body, counter=counter),
        out_shape=[
            jax.ShapeDtypeStruct((_NUM_EMB, _GROUP_DIM), jnp.float32),
            jax.ShapeDtypeStruct((_NUM_EMB, 1), jnp.float32),
            jax.ShapeDtypeStruct((_NUM_EMB, _GROUP_DIM), jnp.float32),
            jax.ShapeDtypeStruct((1, 128), jnp.float32),
        ],
    )(dwx, hc, hdw, dsum)


def _stat_body(dwx_ref, dsum_ref, stat_ref):
    counts = dwx_ref[...][:, _GROUP_DIM:_GROUP_DIM + 1]
    avg_p = counts / float(_ROWS)
    perp = jnp.exp(-jnp.sum(avg_p * jnp.log(avg_p + 1e-06)))
    loss = _CC * (jnp.sum(dsum_ref[...]) / float(_ROWS * _GROUP_DIM))
    lane = lax.broadcasted_iota(jnp.int32, (1, 128), 1)
    stat_ref[...] = jnp.where(lane == 0, loss,
                              jnp.where(lane == 1, perp, 0.0))


def _stat_call(dwx, dsum):
    return pl.pallas_call(
        _stat_body,
        out_shape=[jax.ShapeDtypeStruct((1, 128), jnp.float32)],
    )(dwx, dsum)


def kernel(x, embeddings):
    x4 = x.reshape(_ROWS, _GROUP_NUM, _GROUP_DIM)
    emb = embeddings
    hc = jnp.zeros((_NUM_EMB, 1), jnp.float32)
    hdw = jnp.zeros((_NUM_EMB, _GROUP_DIM), jnp.float32)
    quants, stats = [], []
    for g in range(_GROUP_NUM):
        idx3, dsum, dwx = _argmin_call(g, x4, emb)
        idx2 = idx3.reshape(_NW, _NCHUNK, _CHUNK)
        quant = _gather_kernel()(idx2, emb)[0]
        if g == _GROUP_NUM - 1:
            st = _stat_call(dwx, dsum)[0]
        else:
            emb, hc, hdw, st = _ema_call(g + 1, dwx, hc, hdw, dsum)
        quants.append(quant.reshape(32, 576, _GROUP_DIM))
        stats.append(st)
    quantized = jnp.concatenate(quants, axis=-1)
    loss = sum(st[0, 0] for st in stats) / _GROUP_NUM
    perplexity = sum(st[0, 1] for st in stats) / _GROUP_NUM
    return (quantized, loss, perplexity)


# speedup vs baseline: 1.1497x; 1.1497x over previous
import functools

import jax
import jax.numpy as jnp
from jax import lax
from jax.experimental import pallas as pl
from jax.experimental.pallas import tpu as pltpu
from jax.experimental.pallas import tpu_sc as plsc

_NUM_EMB = 1024
_EMB_DIM = 256
_GROUP_NUM = 4
_GROUP_DIM = _EMB_DIM // _GROUP_NUM
_CC = 0.25
_DECAY = 0.99
_EPS = 1e-05

_ROWS = 32 * 576
_TILE = 2048
_NTILE = _ROWS // _TILE

_NC = 2
_NS = 16
_NW = _NC * _NS
_RPW = _ROWS // _NW
_CHUNK = 64
_NCHUNK = _RPW // _CHUNK


def _argmin_body(flat_ref, emb_ref, idx_ref, dsum_ref, dwx_ref, *, g):
    f = flat_ref[:, g, :]
    e = emb_ref[...]
    fb = f.astype(jnp.bfloat16)
    eb = e.astype(jnp.bfloat16)
    dot = lax.dot_general(fb, eb, (((1,), (1,)), ((), ())),
                          preferred_element_type=jnp.float32)
    f2 = jnp.sum(f * f, axis=1, keepdims=True)
    e2 = jnp.sum(e * e, axis=1)
    d = f2 + e2[None, :] - 2.0 * dot
    m = jnp.min(d, axis=1, keepdims=True)
    col = lax.broadcasted_iota(jnp.int32, d.shape, 1)
    idx = jnp.min(jnp.where(d == m, col, _NUM_EMB), axis=1)
    idx_ref[...] = idx[None, None, :].astype(jnp.int32)

    row = lax.broadcasted_iota(jnp.int32, (_NUM_EMB, _TILE), 0)
    encT = (row == idx[None, :]).astype(jnp.bfloat16)
    fx = jnp.concatenate(
        [fb, jnp.ones((_TILE, _GROUP_DIM), jnp.bfloat16)], axis=1)
    stat = lax.dot_general(encT, fx, (((1,), (0,)), ((), ())),
                           preferred_element_type=jnp.float32)

    part = jnp.sum(m.reshape(_TILE // 128, 128), axis=0)

    @pl.when(pl.program_id(0) == 0)
    def _init():
        dsum_ref[...] = jnp.zeros_like(dsum_ref)
        dwx_ref[...] = jnp.zeros_like(dwx_ref)

    dsum_ref[...] += part[None, :]
    dwx_ref[...] += stat


def _argmin_call(g, x4, emb):
    return pl.pallas_call(
        functools.partial(_argmin_body, g=g),
        grid=(_NTILE,),
        in_specs=[
            pl.BlockSpec((_TILE, _GROUP_NUM, _GROUP_DIM), lambda i: (i, 0, 0)),
            pl.BlockSpec((_NUM_EMB, _GROUP_DIM), lambda i: (0, 0)),
        ],
        out_specs=[
            pl.BlockSpec((1, 1, _TILE), lambda i: (i, 0, 0)),
            pl.BlockSpec((1, 128), lambda i: (0, 0)),
            pl.BlockSpec((_NUM_EMB, 2 * _GROUP_DIM), lambda i: (0, 0)),
        ],
        out_shape=[
            jax.ShapeDtypeStruct((_NTILE, 1, _TILE), jnp.int32),
            jax.ShapeDtypeStruct((1, 128), jnp.float32),
            jax.ShapeDtypeStruct((_NUM_EMB, 2 * _GROUP_DIM), jnp.float32),
        ],
    )(x4, emb)


def _gather_body(idx_hbm, emb_hbm, quant_hbm, idx_v, qrows, sem):
    c = lax.axis_index("c")
    s = lax.axis_index("s")
    wid = c * _NS + s
    base = pl.multiple_of(wid * _RPW, _RPW)

    pltpu.sync_copy(idx_hbm.at[wid], idx_v)
    cps = [
        pltpu.async_copy(emb_hbm.at[idx_v.at[j]],
                         qrows.at[pl.ds(j * _CHUNK, _CHUNK)], sem)
        for j in range(_NCHUNK)
    ]
    for cp in cps:
        cp.wait()
    pltpu.sync_copy(qrows, quant_hbm.at[pl.ds(base, _RPW)])


@functools.cache
def _gather_kernel():
    return pl.kernel(
        _gather_body,
        out_type=[
            jax.ShapeDtypeStruct((_ROWS, _GROUP_DIM), jnp.float32),
        ],
        mesh=plsc.VectorSubcoreMesh(core_axis_name="c", subcore_axis_name="s",
                                    num_cores=_NC, num_subcores=_NS),
        compiler_params=pltpu.CompilerParams(use_tc_tiling_on_sc=False),
        scratch_types=[
            pltpu.VMEM((_NCHUNK, _CHUNK), jnp.int32),
            pltpu.VMEM((_RPW, _GROUP_DIM), jnp.float32),
            pltpu.SemaphoreType.DMA,
        ],
    )


def _ema_body(dwx_ref, hc_ref, hdw_ref, dsum_ref,
              emb_out, hc_out, hdw_out, stat_ref, *, counter):
    dwx = dwx_ref[...]
    counts = dwx[:, _GROUP_DIM:_GROUP_DIM + 1]
    dw = dwx[:, :_GROUP_DIM]
    hc = hc_ref[...]
    hc_new = hc - (hc - counts) * (1.0 - _DECAY)
    bias = 1.0 - _DECAY ** counter
    avg_c = hc_new / bias
    n = jnp.sum(avg_c)
    upd_c = (avg_c + _EPS) / (n + _NUM_EMB * _EPS) * n

    hdw = hdw_ref[...]
    hdw_new = hdw - (hdw - dw) * (1.0 - _DECAY)
    avg_dw = hdw_new / bias

    emb_out[...] = avg_dw / upd_c
    hc_out[...] = hc_new
    hdw_out[...] = hdw_new

    avg_p = counts / float(_ROWS)
    perp = jnp.exp(-jnp.sum(avg_p * jnp.log(avg_p + 1e-06)))
    loss = _CC * (jnp.sum(dsum_ref[...]) / float(_ROWS * _GROUP_DIM))
    lane = lax.broadcasted_iota(jnp.int32, (1, 128), 1)
    stat_ref[...] = jnp.where(lane == 0, loss,
                              jnp.where(lane == 1, perp, 0.0))


def _ema_call(counter, dwx, hc, hdw, dsum):
    return pl.pallas_call(
        functools.partial(_ema_body, counter=counter),
        out_shape=[
            jax.ShapeDtypeStruct((_NUM_EMB, _GROUP_DIM), jnp.float32),
            jax.ShapeDtypeStruct((_NUM_EMB, 1), jnp.float32),
            jax.ShapeDtypeStruct((_NUM_EMB, _GROUP_DIM), jnp.float32),
            jax.ShapeDtypeStruct((1, 128), jnp.float32),
        ],
    )(dwx, hc, hdw, dsum)


def _stat_body(dwx_ref, dsum_ref, stat_ref):
    counts = dwx_ref[...][:, _GROUP_DIM:_GROUP_DIM + 1]
    avg_p = counts / float(_ROWS)
    perp = jnp.exp(-jnp.sum(avg_p * jnp.log(avg_p + 1e-06)))
    loss = _CC * (jnp.sum(dsum_ref[...]) / float(_ROWS * _GROUP_DIM))
    lane = lax.broadcasted_iota(jnp.int32, (1, 128), 1)
    stat_ref[...] = jnp.where(lane == 0, loss,
                              jnp.where(lane == 1, perp, 0.0))


def _stat_call(dwx, dsum):
    return pl.pallas_call(
        _stat_body,
        out_shape=[jax.ShapeDtypeStruct((1, 128), jnp.float32)],
    )(dwx, dsum)


def kernel(x, embeddings):
    x4 = x.reshape(_ROWS, _GROUP_NUM, _GROUP_DIM)
    emb = embeddings
    hc = jnp.zeros((_NUM_EMB, 1), jnp.float32)
    hdw = jnp.zeros((_NUM_EMB, _GROUP_DIM), jnp.float32)
    quants, stats = [], []
    for g in range(_GROUP_NUM):
        idx3, dsum, dwx = _argmin_call(g, x4, emb)
        idx2 = idx3.reshape(_NW, _NCHUNK, _CHUNK)
        quant = _gather_kernel()(idx2, emb)[0]
        if g == _GROUP_NUM - 1:
            st = _stat_call(dwx, dsum)[0]
        else:
            emb, hc, hdw, st = _ema_call(g + 1, dwx, hc, hdw, dsum)
        quants.append(quant.reshape(32, 576, _GROUP_DIM))
        stats.append(st)
    quantized = jnp.concatenate(quants, axis=-1)
    loss = sum(st[0, 0] for st in stats) / _GROUP_NUM
    perplexity = sum(st[0, 1] for st in stats) / _GROUP_NUM
    return (quantized, loss, perplexity)
